# baseline (speedup 1.0000x reference)
import functools

import jax
import jax.numpy as jnp
from jax import lax
from jax.experimental import pallas as pl
from jax.experimental.pallas import tpu as pltpu
from jax.experimental.pallas import tpu_sc as plsc

B = 16384
D = 64
NC = 2
NS = 16
NW = NC * NS
BPW = B // NW


def _body(h_hbm, ent_hbm, out_hbm, ib, vb, ob, sem):
    c = lax.axis_index("c")
    s = lax.axis_index("s")
    lid = c * NS + s
    iota = lax.iota(jnp.int32, 16)
    pltpu.sync_copy(h_hbm.at[pl.ds(lid * BPW, BPW)], ib)

    def row(i, _):
        hv = ib[pl.ds(0, 16)]
        base = hv * 0 + iota * 1000000
        vb2 = base + i
        ib[pl.ds(16, 16)] = vb2
        return _

    lax.fori_loop(0, 4, row, None)

    def gath(g, _):
        cp = pltpu.async_copy(ent_hbm.at[ib.at[pl.ds(0, 128)]],
                              vb.at[pl.ds(0, 128)], sem)
        cp.wait()
        return _

    lax.fori_loop(0, 4, gath, None)

    def comb(i, _):
        for k in range(D // 16):
            sl = pl.ds(k * 16, 16)
            ob[i, sl] = vb[pl.ds(i * 16, 16)] * 1.0 + ob[i, sl]
        return _

    lax.fori_loop(0, 128, comb, None)
    pltpu.sync_copy(ob, out_hbm.at[pl.ds(lid * 128, 128)])


@jax.jit
def kernel(h, r, t, entity_table, relation_table):
    mesh = plsc.VectorSubcoreMesh(core_axis_name="c", subcore_axis_name="s")
    k = functools.partial(
        pl.kernel,
        mesh=mesh,
        compiler_params=pltpu.CompilerParams(use_tc_tiling_on_sc=False),
        out_type=jax.ShapeDtypeStruct((B // 4, D), jnp.float32),
        scratch_types=[
            pltpu.VMEM((BPW,), jnp.int32),
            pltpu.VMEM((8192,), jnp.float32),
            pltpu.VMEM((128, D), jnp.float32),
            pltpu.SemaphoreType.DMA,
        ],
    )(_body)
    ent_flat = entity_table.T.reshape(-1)
    o = k(h, ent_flat)
    return jnp.tile(o, (4, 1)) * 0.0


# speedup vs baseline: 11.4472x; 11.4472x over previous
import functools

import jax
import jax.numpy as jnp
from jax import lax
from jax.experimental import pallas as pl
from jax.experimental.pallas import tpu as pltpu
from jax.experimental.pallas import tpu_sc as plsc

B = 16384
D = 64
NC = 2
NS = 16
NW = NC * NS
BPW = B // NW
CHUNK = 32
NCHUNK = BPW // CHUNK


def _body(h_hbm, r_hbm, t_hbm, ent_hbm, rel_hbm, out_hbm,
          hi, ti, ri, hb, tb, rbuf, ob, sem, sem_r):
    c = lax.axis_index("c")
    s = lax.axis_index("s")
    lid = c * NS + s
    base = lid * BPW
    pltpu.sync_copy(h_hbm.at[pl.ds(base, BPW)], hi)
    pltpu.sync_copy(t_hbm.at[pl.ds(base, BPW)], ti)
    pltpu.sync_copy(r_hbm.at[pl.ds(lid * NCHUNK, NCHUNK)], ri)

    def chunk(ci, _):
        rel_cp = pltpu.async_copy(rel_hbm.at[ri.at[ci]], rbuf, sem_r)

        def fire(g, _):
            hv = hi[pl.ds(ci * CHUNK + g * 16, 16)]
            tv = ti[pl.ds(ci * CHUNK + g * 16, 16)]
            for l in range(16):
                bh = pl.multiple_of((hv[l] >> 3) << 3, 8)
                bt = pl.multiple_of((tv[l] >> 3) << 3, 8)
                pltpu.async_copy(ent_hbm.at[pl.ds(bh, 8), :],
                                 hb.at[g * 16 + l], sem)
                pltpu.async_copy(ent_hbm.at[pl.ds(bt, 8), :],
                                 tb.at[g * 16 + l], sem)
            return _

        lax.fori_loop(0, CHUNK // 16, fire, None)

        def drain(g, _):
            for l in range(16):
                pltpu.make_async_copy(ent_hbm.at[pl.ds(0, 8), :],
                                      hb.at[g * 16 + l], sem).wait()
                pltpu.make_async_copy(ent_hbm.at[pl.ds(0, 8), :],
                                      tb.at[g * 16 + l], sem).wait()
            return _

        lax.fori_loop(0, CHUNK // 16, drain, None)
        rel_cp.wait()

        def comb(g, _):
            hv = hi[pl.ds(ci * CHUNK + g * 16, 16)]
            tv = ti[pl.ds(ci * CHUNK + g * 16, 16)]
            for l in range(16):
                i = g * 16 + l
                rh = hv[l] & 7
                rt = tv[l] & 7
                for k in range(D // 16):
                    sl = pl.ds(k * 16, 16)
                    ob[i, sl] = hb[i, rh, sl] + rbuf[i, sl] - tb[i, rt, sl]
            return _

        lax.fori_loop(0, CHUNK // 16, comb, None)
        pltpu.sync_copy(ob, out_hbm.at[pl.ds(base + ci * CHUNK, CHUNK)])
        return _

    lax.fori_loop(0, NCHUNK, chunk, None)


@jax.jit
def kernel(h, r, t, entity_table, relation_table):
    mesh = plsc.VectorSubcoreMesh(core_axis_name="c", subcore_axis_name="s")
    k = functools.partial(
        pl.kernel,
        mesh=mesh,
        out_type=jax.ShapeDtypeStruct((B, D), jnp.float32),
        scratch_types=[
            pltpu.VMEM((BPW,), jnp.int32),
            pltpu.VMEM((BPW,), jnp.int32),
            pltpu.VMEM((NCHUNK, CHUNK), jnp.int32),
            pltpu.VMEM((CHUNK, 8, D), jnp.float32),
            pltpu.VMEM((CHUNK, 8, D), jnp.float32),
            pltpu.VMEM((CHUNK, 2 * D), jnp.float32),
            pltpu.VMEM((CHUNK, D), jnp.float32),
            pltpu.SemaphoreType.DMA,
            pltpu.SemaphoreType.DMA,
        ],
    )(_body)
    rel2 = jnp.pad(relation_table, ((0, 0), (0, D)))
    r2 = r.reshape(B // CHUNK, CHUNK)
    return k(h, r2, t, entity_table, rel2)


# speedup vs baseline: 11.4840x; 1.0032x over previous
import functools

import jax
import jax.numpy as jnp
from jax import lax
from jax.experimental import pallas as pl
from jax.experimental.pallas import tpu as pltpu
from jax.experimental.pallas import tpu_sc as plsc

B = 16384
D = 64
NC = 2
NS = 16
NW = NC * NS
BPW = B // NW
CHUNK = 32
NCHUNK = BPW // CHUNK


def _body(h_hbm, r_hbm, t_hbm, ent_hbm, rel_hbm, out_hbm,
          hi, ti, ri, hb, tb, rbuf, ob, sem, sem_r):
    c = lax.axis_index("c")
    s = lax.axis_index("s")
    lid = c * NS + s
    base = lid * BPW
    pltpu.sync_copy(h_hbm.at[pl.ds(base, BPW)], hi)
    pltpu.sync_copy(t_hbm.at[pl.ds(base, BPW)], ti)
    pltpu.sync_copy(r_hbm.at[pl.ds(lid * NCHUNK, NCHUNK)], ri)

    def chunk(ci, _):
        rel_cp = pltpu.async_copy(rel_hbm.at[ri.at[ci]], rbuf, sem_r)

        def fire(g, _):
            hv = (hi[pl.ds(ci * CHUNK + g * 16, 16)] >> 3) << 3
            tv = (ti[pl.ds(ci * CHUNK + g * 16, 16)] >> 3) << 3
            for l in range(16):
                bh = pl.multiple_of(hv[l], 8)
                bt = pl.multiple_of(tv[l], 8)
                pltpu.async_copy(ent_hbm.at[pl.ds(bh, 8), :],
                                 hb.at[g * 16 + l], sem)
                pltpu.async_copy(ent_hbm.at[pl.ds(bt, 8), :],
                                 tb.at[g * 16 + l], sem)
            return _

        lax.fori_loop(0, CHUNK // 16, fire, None)

        def drain(g, _):
            for l in range(16):
                pltpu.make_async_copy(ent_hbm.at[pl.ds(0, 8), :],
                                      hb.at[g * 16 + l], sem).wait()
                pltpu.make_async_copy(ent_hbm.at[pl.ds(0, 8), :],
                                      tb.at[g * 16 + l], sem).wait()
            return _

        lax.fori_loop(0, CHUNK // 16, drain, None)
        rel_cp.wait()

        def comb(g, _):
            hv = hi[pl.ds(ci * CHUNK + g * 16, 16)]
            tv = ti[pl.ds(ci * CHUNK + g * 16, 16)]
            for l in range(16):
                i = g * 16 + l
                rh = hv[l] & 7
                rt = tv[l] & 7
                for k in range(D // 16):
                    sl = pl.ds(k * 16, 16)
                    ob[i, sl] = hb[i, rh, sl] + rbuf[i, sl] - tb[i, rt, sl]
            return _

        lax.fori_loop(0, CHUNK // 16, comb, None)
        pltpu.sync_copy(ob, out_hbm.at[pl.ds(base + ci * CHUNK, CHUNK)])
        return _

    lax.fori_loop(0, NCHUNK, chunk, None)


@jax.jit
def kernel(h, r, t, entity_table, relation_table):
    mesh = plsc.VectorSubcoreMesh(core_axis_name="c", subcore_axis_name="s")
    k = functools.partial(
        pl.kernel,
        mesh=mesh,
        out_type=jax.ShapeDtypeStruct((B, D), jnp.float32),
        scratch_types=[
            pltpu.VMEM((BPW,), jnp.int32),
            pltpu.VMEM((BPW,), jnp.int32),
            pltpu.VMEM((NCHUNK, CHUNK), jnp.int32),
            pltpu.VMEM((CHUNK, 8, D), jnp.float32),
            pltpu.VMEM((CHUNK, 8, D), jnp.float32),
            pltpu.VMEM((CHUNK, 2 * D), jnp.float32),
            pltpu.VMEM((CHUNK, D), jnp.float32),
            pltpu.SemaphoreType.DMA,
            pltpu.SemaphoreType.DMA,
        ],
    )(_body)
    rel2 = jnp.pad(relation_table, ((0, 0), (0, D)))
    r2 = r.reshape(B // CHUNK, CHUNK)
    return k(h, r2, t, entity_table, rel2)
